# Initial kernel scaffold; baseline (speedup 1.0000x reference)
import jax
import jax.numpy as jnp
from jax.experimental import pallas as pl


def kernel(src_tokens, embed_tokens, pos_emb, ln_gamma, ln_beta):
    raise NotImplementedError("write your pallas kernel here")


# speedup vs baseline: 3.3023x; 3.3023x over previous
import functools
import math

import jax
import jax.numpy as jnp
from jax import lax
from jax.experimental import pallas as pl
from jax.experimental.pallas import tpu as pltpu
from jax.experimental.pallas import tpu_sc as plsc

_VOCAB = 100000
_D = 64
_PAD = 1
_B = 4096
_S = 200
_SPAD = 208
_NC = 2
_NS = 16
_NW = _NC * _NS
_ROWS_PER_W = _B // _NW
_SCALE = math.sqrt(_D)


def _rsqrt16(a):
    i = lax.bitcast_convert_type(a, jnp.int32)
    y = lax.bitcast_convert_type(jnp.int32(0x5F3759DF) - (i >> 1), jnp.float32)
    half = 0.5 * a
    for _ in range(3):
        y = y * (1.5 - half * y * y)
    return y


def _shuffle(v, idx):
    return v.at[idx].get(mode="promise_in_bounds")


def _lane_sum(v, lane):
    for d in (8, 4, 2, 1):
        v = v + _shuffle(v, lane ^ d)
    return v


def _sc_body(src_hbm, tok_hbm, pos_hbm, g_hbm, b_hbm, out_hbm,
             srcb, posidx, tokrows, posrows, outb, g_v, b_v, sem_t, sem_p):
    wid = lax.axis_index("s") * _NC + lax.axis_index("c")
    base = wid * _ROWS_PER_W

    pltpu.sync_copy(g_hbm, g_v)
    pltpu.sync_copy(b_hbm, b_v)
    gs = [g_v[pl.ds(16 * k, 16)] for k in range(4)]
    bs = [b_v[pl.ds(16 * k, 16)] for k in range(4)]
    lane = lax.iota(jnp.int32, 16)
    lane15 = lane | 15

    def row_body(j, carry_unused):
        row = base + j
        pltpu.sync_copy(src_hbm.at[pl.ds(row * _S, _S)], srcb.at[pl.ds(0, _S)])

        carry = jnp.zeros((16,), jnp.int32)
        for i in range(_SPAD // 16):
            v = srcb[pl.ds(16 * i, 16)]
            m = v != _PAD
            if i == (_SPAD // 16) - 1:
                tail_ok = lane < (_S - 16 * i)
                m = m & tail_ok
                srcb[pl.ds(16 * i, 16)] = jnp.where(tail_ok, v, 0)
            mi = jnp.where(m, jnp.int32(1), jnp.int32(0))
            c = jnp.cumsum(mi) + carry
            posidx[pl.ds(16 * i, 16)] = jnp.where(m, c + _PAD, _PAD)
            carry = _shuffle(c, lane15)

        cp_t = pltpu.async_copy(tok_hbm.at[srcb], tokrows, sem_t)
        cp_p = pltpu.async_copy(pos_hbm.at[posidx], posrows, sem_p)
        cp_t.wait()
        cp_p.wait()

        def ln_body(t, carry2):
            x = [_SCALE * tokrows[t, pl.ds(16 * k, 16)]
                 + posrows[t, pl.ds(16 * k, 16)] for k in range(4)]
            mu = _lane_sum(x[0] + x[1] + x[2] + x[3], lane) * (1.0 / _D)
            c = [xk - mu for xk in x]
            var = _lane_sum(c[0] * c[0] + c[1] * c[1]
                            + c[2] * c[2] + c[3] * c[3], lane) * (1.0 / _D)
            r = _rsqrt16(var + 1e-5)
            for k in range(4):
                outb[t, pl.ds(16 * k, 16)] = c[k] * r * gs[k] + bs[k]
            return carry2

        lax.fori_loop(0, _S, ln_body, 0)
        pltpu.sync_copy(outb.at[pl.ds(0, _S)], out_hbm.at[pl.ds(row * _S, _S)])
        return carry_unused

    lax.fori_loop(0, _ROWS_PER_W, row_body, 0)


def kernel(src_tokens, embed_tokens, pos_emb, ln_gamma, ln_beta):
    mesh = plsc.VectorSubcoreMesh(core_axis_name="c", subcore_axis_name="s")
    f = functools.partial(
        pl.kernel,
        mesh=mesh,
        compiler_params=pltpu.CompilerParams(use_tc_tiling_on_sc=False,
                                             needs_layout_passes=False),
        out_type=jax.ShapeDtypeStruct((_B * _S, _D), jnp.float32),
        scratch_types=[
            pltpu.VMEM((_SPAD,), jnp.int32),
            pltpu.VMEM((_SPAD,), jnp.int32),
            pltpu.VMEM((_SPAD, _D), jnp.float32),
            pltpu.VMEM((_SPAD, _D), jnp.float32),
            pltpu.VMEM((_SPAD, _D), jnp.float32),
            pltpu.VMEM((_D,), jnp.float32),
            pltpu.VMEM((_D,), jnp.float32),
            pltpu.SemaphoreType.DMA,
            pltpu.SemaphoreType.DMA,
        ],
    )(_sc_body)
    out = f(src_tokens.reshape(_B * _S), embed_tokens, pos_emb,
            ln_gamma, ln_beta)
    return out.reshape(_B, _S, _D)
